# baseline (speedup 1.0000x reference)
import functools

import jax
import jax.numpy as jnp
from jax import lax
from jax.experimental import pallas as pl
from jax.experimental.pallas import tpu as pltpu
from jax.experimental.pallas import tpu_sc as plsc

VOCAB = 23236
EMBED_DIM = 1728
SEG = 14
PADDED = SEG * 128
BATCH = 1024
SEQ = 50
B = BATCH * SEQ

NC = 2
NS = 16
NW = NC * NS
BPW = B // NW
CH = 16
NCHUNK = BPW // CH
SUB = 2
NIDX = CH // SUB * SEG
ROWS = CH * SEG
IDXPW = BPW * SEG

_MESH = plsc.VectorSubcoreMesh(core_axis_name="c", subcore_axis_name="s")


@functools.partial(
    pl.kernel,
    out_type=jax.ShapeDtypeStruct((B * SEG, 128), jnp.float32),
    mesh=_MESH,
    compiler_params=pltpu.CompilerParams(use_tc_tiling_on_sc=False),
    scratch_types=[
        pltpu.VMEM((IDXPW,), jnp.int32),
        pltpu.VMEM((2, ROWS, 128), jnp.float32),
        pltpu.SemaphoreType.DMA,
        pltpu.SemaphoreType.DMA,
        pltpu.SemaphoreType.DMA,
    ],
)
def _glyph_gather(idx_hbm, tab_hbm, out_hbm, idx_v, rows_v, gsem, osem0, osem1):
    wid = lax.axis_index("s") * NC + lax.axis_index("c")
    base = wid * IDXPW
    osems = (osem0, osem1)

    pltpu.sync_copy(idx_hbm.at[pl.ds(wid * IDXPW, IDXPW)], idx_v)

    def start_gathers(j, b):
        for q in range(SUB):
            pltpu.async_copy(
                tab_hbm.at[idx_v.at[pl.ds(j * ROWS + q * NIDX, NIDX)]],
                rows_v.at[b, pl.ds(q * NIDX, NIDX)],
                gsem,
            )

    def wait_gathers(b):
        pltpu.make_async_copy(
            tab_hbm.at[pl.ds(0, ROWS)], rows_v.at[b], gsem
        ).wait()

    start_gathers(0, 0)

    def pair(p, carry):
        for b in range(2):
            j = 2 * p + b
            wait_gathers(b)

            @pl.when(j >= 1)
            def _():
                pltpu.make_async_copy(
                    rows_v.at[1 - b], out_hbm.at[pl.ds(base, ROWS)], osems[1 - b]
                ).wait()

            @pl.when(j + 1 < NCHUNK)
            def _():
                start_gathers(j + 1, 1 - b)

            pltpu.async_copy(
                rows_v.at[b], out_hbm.at[pl.ds(base + j * ROWS, ROWS)], osems[b]
            )
        return carry

    lax.fori_loop(0, NCHUNK // 2, pair, 0)
    pltpu.make_async_copy(
        rows_v.at[1], out_hbm.at[pl.ds(base, ROWS)], osem1
    ).wait()


def kernel(input_ids, embedding_table):
    ids = input_ids.reshape(-1).astype(jnp.int32)
    segs = (ids[:, None] * SEG + jnp.arange(SEG, dtype=jnp.int32)).reshape(-1)
    table_seg = jnp.pad(embedding_table, ((0, 0), (0, PADDED - EMBED_DIM)))
    table_seg = table_seg.reshape(VOCAB * SEG, 128)
    out = _glyph_gather(segs, table_seg)
    return out.reshape(B, PADDED)[:, :EMBED_DIM].reshape(BATCH, SEQ, EMBED_DIM)


# speedup vs baseline: 1.4008x; 1.4008x over previous
import functools

import jax
import jax.numpy as jnp
from jax import lax
from jax.experimental import pallas as pl
from jax.experimental.pallas import tpu as pltpu
from jax.experimental.pallas import tpu_sc as plsc

VOCAB = 23236
EMBED_DIM = 1728
SEG = 14
PADDED = SEG * 128
BATCH = 1024
SEQ = 50
B = BATCH * SEQ

NC = 2
NS = 16
NW = NC * NS
BPW = B // NW
CH = 16
NCHUNK = BPW // CH
SUB = 2
NIDX = CH // SUB * SEG
ROWS = CH * SEG
IDXPW = BPW * SEG

_MESH = plsc.VectorSubcoreMesh(core_axis_name="c", subcore_axis_name="s")


@functools.partial(
    pl.kernel,
    out_type=jax.ShapeDtypeStruct((B * SEG, 128), jnp.float32),
    mesh=_MESH,
    compiler_params=pltpu.CompilerParams(use_tc_tiling_on_sc=False),
    scratch_types=[
        pltpu.VMEM((IDXPW,), jnp.int32),
        pltpu.VMEM((2, ROWS, 128), jnp.float32),
        pltpu.SemaphoreType.DMA,
        pltpu.SemaphoreType.DMA,
        pltpu.SemaphoreType.DMA,
    ],
)
def _glyph_gather(idx_hbm, tab_hbm, out_hbm, idx_v, rows_v, gsem, osem0, osem1):
    wid = lax.axis_index("s") * NC + lax.axis_index("c")
    base = wid * IDXPW
    osems = (osem0, osem1)

    pltpu.sync_copy(idx_hbm.at[pl.ds(wid * IDXPW, IDXPW)], idx_v)

    def start_gathers(j, b):
        for q in range(SUB):
            pltpu.async_copy(
                tab_hbm.at[idx_v.at[pl.ds(j * ROWS + q * NIDX, NIDX)]],
                rows_v.at[b, pl.ds(q * NIDX, NIDX)],
                gsem,
            )

    def wait_gathers(b):
        pltpu.make_async_copy(
            tab_hbm.at[pl.ds(0, ROWS)], rows_v.at[b], gsem
        ).wait()

    start_gathers(0, 0)

    def pair(p, carry):
        for b in range(2):
            j = 2 * p + b
            wait_gathers(b)

            @pl.when(j >= 1)
            def _():
                pltpu.make_async_copy(
                    rows_v.at[1 - b], out_hbm.at[pl.ds(base, ROWS)], osems[1 - b]
                ).wait()

            @pl.when(j + 1 < NCHUNK)
            def _():
                start_gathers(j + 1, 1 - b)

            pltpu.async_copy(
                rows_v.at[b], out_hbm.at[pl.ds(base + j * ROWS, ROWS)], osems[b]
            )
        return carry

    lax.fori_loop(0, NCHUNK // 2, pair, 0)
    pltpu.make_async_copy(
        rows_v.at[1], out_hbm.at[pl.ds(base, ROWS)], osem1
    ).wait()


_PAD_RB = 128


def _pad_body(t_ref, o_ref):
    x = t_ref[...]
    y = jnp.pad(x, ((0, 0), (0, PADDED - EMBED_DIM)))
    o_ref[...] = y.reshape(_PAD_RB * SEG, 128)


def _tc_pad(table):
    grid = (VOCAB + _PAD_RB - 1) // _PAD_RB
    return pl.pallas_call(
        _pad_body,
        grid=(grid,),
        in_specs=[pl.BlockSpec((_PAD_RB, EMBED_DIM), lambda i: (i, 0))],
        out_specs=pl.BlockSpec((_PAD_RB * SEG, 128), lambda i: (i, 0)),
        out_shape=jax.ShapeDtypeStruct((VOCAB * SEG, 128), jnp.float32),
    )(table)


_DET_BB = 2


def _detile_body(x_ref, o_ref):
    x = x_ref[...]
    y = x.reshape(_DET_BB * SEQ, PADDED)[:, :EMBED_DIM]
    o_ref[...] = y.reshape(_DET_BB, SEQ, EMBED_DIM)


def _tc_detile(x):
    return pl.pallas_call(
        _detile_body,
        grid=(BATCH // _DET_BB,),
        in_specs=[pl.BlockSpec((_DET_BB * SEQ * SEG, 128), lambda i: (i, 0))],
        out_specs=pl.BlockSpec((_DET_BB, SEQ, EMBED_DIM), lambda i: (i, 0, 0)),
        out_shape=jax.ShapeDtypeStruct((BATCH, SEQ, EMBED_DIM), jnp.float32),
    )(x)


def kernel(input_ids, embedding_table):
    ids = input_ids.reshape(-1).astype(jnp.int32)
    segs = (ids[:, None] * SEG + jnp.arange(SEG, dtype=jnp.int32)).reshape(-1)
    table_seg = _tc_pad(embedding_table)
    out = _glyph_gather(segs, table_seg)
    return _tc_detile(out)
